# baseline (speedup 1.0000x reference)
import functools

import jax
import jax.numpy as jnp
from jax import lax
from jax.experimental import pallas as pl
from jax.experimental.pallas import tpu as pltpu
from jax.experimental.pallas import tpu_sc as plsc

NUM_EMB = 1000000
DIM = 32
BATCH = 16384
BLK = 128
WIN = 128

_info = plsc.get_sparse_core_info()
_NC, _NS, _L = _info.num_cores, _info.num_subcores, _info.num_lanes
_NW = _NC * _NS
_BPW = BATCH // _NW
_GROUPS = _BPW // _L
_H = _L // 2


def _sc_lookup(tin_hbm, idx_hbm, out_hbm, idx_v, blk_v, out_v, sem_a, sem_b):
    wid = lax.axis_index("s") * _NC + lax.axis_index("c")
    base = wid * _BPW

    pltpu.sync_copy(idx_hbm.at[pl.ds(base, _BPW)], idx_v)

    iota = lax.iota(jnp.int32, _L)
    mask_a = iota < _H
    mask_b = iota >= _H
    jvecs = [jnp.full((_L,), j, jnp.int32) for j in range(DIM)]

    def fire(chunk, lanes, sem):
        cp = None
        for lane in lanes:
            ic = jnp.minimum(chunk[lane], NUM_EMB - 1)
            wstart = pl.multiple_of((ic >> 7) << 7, BLK)
            for t in range(4):
                cp = pltpu.async_copy(
                    tin_hbm.at[pl.ds(8 * t, 8), pl.ds(wstart, WIN)],
                    blk_v.at[lane, pl.ds(8 * t, 8)],
                    sem,
                )
        return cp

    def extract(g, rvec, mask):
        kvec = iota + g * _L
        for j in range(DIM):
            val = plsc.load_gather(blk_v, [iota, jvecs[j], rvec])
            plsc.store_scatter(out_v, [jvecs[j], kvec], val, mask=mask)

    def _wait_one(sem):
        for t in range(4):
            pltpu.make_async_copy(
                tin_hbm.at[pl.ds(8 * t, 8), pl.ds(0, WIN)],
                blk_v.at[0, pl.ds(8 * t, 8)],
                sem,
            ).wait()

    def loop_body(p, bad):
        chunk = idx_v[pl.ds(p * _L, _L)]
        bad = jnp.maximum(bad, (chunk == NUM_EMB).astype(jnp.int32))
        rvec = jnp.minimum(chunk, NUM_EMB - 1) & (BLK - 1)

        cpb = fire(chunk, range(_H, _L), sem_b)
        for _ in range(_H):
            _wait_one(sem_a)
        extract(p, rvec, mask_a)

        @pl.when(p < _GROUPS - 1)
        def _():
            nxt = idx_v[pl.ds((p + 1) * _L, _L)]
            fire(nxt, range(0, _H), sem_a)

        for _ in range(4 * _H):
            cpb.wait()
        extract(p, rvec, mask_b)
        return bad

    chunk0 = idx_v[pl.ds(0, _L)]
    fire(chunk0, range(0, _H), sem_a)

    bad = lax.fori_loop(
        0, _GROUPS, loop_body, jnp.zeros((_L,), jnp.int32)
    )
    haspad = jnp.max(bad)

    @pl.when(haspad > 0)
    def _():
        zeros16 = jnp.zeros((_L,), jnp.float32)
        for g in range(_GROUPS):
            v = idx_v[pl.ds(g * _L, _L)]
            pad = v == NUM_EMB
            kvec = lax.iota(jnp.int32, _L) + g * _L
            for j in range(DIM):
                plsc.store_scatter(out_v, [jvecs[j], kvec], zeros16, mask=pad)

    pltpu.sync_copy(out_v, out_hbm.at[:, pl.ds(base, _BPW)])


@jax.jit
def kernel(embeddings, indices):
    idx = indices.astype(jnp.int32)
    table_t = embeddings.T
    mesh = plsc.VectorSubcoreMesh(core_axis_name="c", subcore_axis_name="s")
    f = functools.partial(
        pl.kernel,
        out_type=jax.ShapeDtypeStruct((DIM, BATCH), jnp.float32),
        mesh=mesh,
        scratch_types=[
            pltpu.VMEM((_BPW,), jnp.int32),
            pltpu.VMEM((_L, DIM, WIN), jnp.float32),
            pltpu.VMEM((DIM, _BPW), jnp.float32),
            pltpu.SemaphoreType.DMA,
            pltpu.SemaphoreType.DMA,
        ],
        compiler_params=pltpu.CompilerParams(needs_layout_passes=False),
    )(_sc_lookup)
    return f(table_t, idx).T


# speedup vs baseline: 1.0093x; 1.0093x over previous
import functools

import jax
import jax.numpy as jnp
from jax import lax
from jax.experimental import pallas as pl
from jax.experimental.pallas import tpu as pltpu
from jax.experimental.pallas import tpu_sc as plsc

NUM_EMB = 1000000
DIM = 32
BATCH = 16384
BLK = 128
WIN = 128

_info = plsc.get_sparse_core_info()
_NC, _NS, _L = _info.num_cores, _info.num_subcores, _info.num_lanes
_NW = _NC * _NS
_BPW = BATCH // _NW
_GROUPS = _BPW // _L
_H = _L // 2


def _sc_lookup(tin_hbm, idx_hbm, out_hbm, idx_v, blk_v, out_v, sem_a, sem_b):
    wid = lax.axis_index("s") * _NC + lax.axis_index("c")
    base = wid * _BPW

    pltpu.sync_copy(idx_hbm.at[pl.ds(base, _BPW)], idx_v)

    iota = lax.iota(jnp.int32, _L)
    mask_a = iota < _H
    mask_b = iota >= _H
    jvecs = [jnp.full((_L,), j, jnp.int32) for j in range(DIM)]

    def fire(chunk, lanes, sem):
        cp = None
        for lane in lanes:
            ic = jnp.minimum(chunk[lane], NUM_EMB - 1)
            wstart = pl.multiple_of((ic >> 7) << 7, BLK)
            cp = pltpu.async_copy(
                tin_hbm.at[:, pl.ds(wstart, WIN)], blk_v.at[lane], sem
            )
        return cp

    def extract(g, rvec, mask):
        kvec = iota + g * _L
        for j in range(DIM):
            val = plsc.load_gather(blk_v, [iota, jvecs[j], rvec])
            plsc.store_scatter(out_v, [jvecs[j], kvec], val, mask=mask)

    def _wait_one(sem):
        pltpu.make_async_copy(
            tin_hbm.at[:, pl.ds(0, WIN)], blk_v.at[0], sem
        ).wait()

    def loop_body(p, bad):
        chunk = idx_v[pl.ds(p * _L, _L)]
        bad = jnp.maximum(bad, (chunk == NUM_EMB).astype(jnp.int32))
        rvec = jnp.minimum(chunk, NUM_EMB - 1) & (BLK - 1)

        cpb = fire(chunk, range(_H, _L), sem_b)
        for _ in range(_H):
            _wait_one(sem_a)
        extract(p, rvec, mask_a)

        @pl.when(p < _GROUPS - 1)
        def _():
            nxt = idx_v[pl.ds((p + 1) * _L, _L)]
            fire(nxt, range(0, _H), sem_a)

        for _ in range(_H):
            cpb.wait()
        extract(p, rvec, mask_b)
        return bad

    chunk0 = idx_v[pl.ds(0, _L)]
    fire(chunk0, range(0, _H), sem_a)

    bad = lax.fori_loop(
        0, _GROUPS, loop_body, jnp.zeros((_L,), jnp.int32)
    )
    haspad = jnp.max(bad)

    @pl.when(haspad > 0)
    def _():
        zeros16 = jnp.zeros((_L,), jnp.float32)
        for g in range(_GROUPS):
            v = idx_v[pl.ds(g * _L, _L)]
            pad = v == NUM_EMB
            kvec = lax.iota(jnp.int32, _L) + g * _L
            for j in range(DIM):
                plsc.store_scatter(out_v, [jvecs[j], kvec], zeros16, mask=pad)

    pltpu.sync_copy(out_v, out_hbm.at[:, pl.ds(base, _BPW)])


@jax.jit
def kernel(embeddings, indices):
    idx = indices.astype(jnp.int32)
    table_t = embeddings.T
    mesh = plsc.VectorSubcoreMesh(core_axis_name="c", subcore_axis_name="s")
    f = functools.partial(
        pl.kernel,
        out_type=jax.ShapeDtypeStruct((DIM, BATCH), jnp.float32),
        mesh=mesh,
        scratch_types=[
            pltpu.VMEM((_BPW,), jnp.int32),
            pltpu.VMEM((_L, DIM, WIN), jnp.float32),
            pltpu.VMEM((DIM, _BPW), jnp.float32),
            pltpu.SemaphoreType.DMA,
            pltpu.SemaphoreType.DMA,
        ],
        compiler_params=pltpu.CompilerParams(needs_layout_passes=False),
    )(_sc_lookup)
    return f(table_t, idx).T
